# Initial kernel scaffold; baseline (speedup 1.0000x reference)
import jax
import jax.numpy as jnp
from jax.experimental import pallas as pl


def kernel(feature, xyz, neigh_idx, params):
    raise NotImplementedError("write your pallas kernel here")


# speedup vs baseline: 8.2460x; 8.2460x over previous
import functools

import jax
import jax.numpy as jnp
from jax import lax
from jax.experimental import pallas as pl
from jax.experimental.pallas import tpu as pltpu
from jax.experimental.pallas import tpu_sc as plsc

EPS = 1e-5
LAMDA = 1.0


def _sc_gather(table, idx_flat):
    info = plsc.get_sparse_core_info()
    nw = info.num_cores * info.num_subcores
    m, d = idx_flat.shape[0], table.shape[1]
    per_w = m // nw
    chunk = 2048
    n_ch = per_w // chunk
    mesh = plsc.VectorSubcoreMesh(core_axis_name="c", subcore_axis_name="s")

    @functools.partial(
        pl.kernel, mesh=mesh,
        compiler_params=pltpu.CompilerParams(use_tc_tiling_on_sc=False),
        out_type=jax.ShapeDtypeStruct((m, d), jnp.float32),
        scratch_types=[
            pltpu.VMEM((chunk,), jnp.int32),
            pltpu.VMEM((chunk, d), jnp.float32),
            pltpu.SemaphoreType.DMA,
        ],
    )
    def k(table_hbm, idx_hbm, out_hbm, idx_v, rows_v, sem):
        wid = lax.axis_index("s") * info.num_cores + lax.axis_index("c")
        base = wid * per_w

        def body(i, carry):
            off = base + i * chunk
            pltpu.sync_copy(idx_hbm.at[pl.ds(off, chunk)], idx_v)
            pltpu.async_copy(table_hbm.at[idx_v], rows_v, sem).wait()
            pltpu.sync_copy(rows_v, out_hbm.at[pl.ds(off, chunk)])
            return carry

        lax.fori_loop(0, n_ch, body, 0)

    return k(table, idx_flat)


def _mm(x, w):
    return jnp.dot(x, w.T, preferred_element_type=jnp.float32)


def _mm3(x, w):
    s = x.shape
    y = jnp.dot(x.reshape(s[0] * s[1], s[2]), w.T,
                preferred_element_type=jnp.float32)
    return y.reshape(s[0], s[1], w.shape[0])


def _stat2(y):
    axes = tuple(range(y.ndim - 1))
    s = jnp.sum(y, axis=axes)
    q = jnp.sum(y * y, axis=axes)
    z = jnp.zeros((6, y.shape[-1]), jnp.float32)
    return jnp.concatenate([s[None], q[None], z], axis=0)


def _bn_apply(y, pack, relu=True):
    sc = pack[0:1]
    sh = pack[1:2]
    if y.ndim == 3:
        sc, sh = sc[None], sh[None]
    y = y * sc + sh
    return jnp.maximum(y, 0.0) if relu else y


def _finalize(st, gamma, beta, cnt):
    m = st[0] / cnt
    v = st[1] / cnt - m * m
    sc = gamma / jnp.sqrt(v + EPS)
    sh = beta - m * sc
    return jnp.concatenate(
        [sc[None], sh[None], jnp.zeros((6, sc.shape[0]), jnp.float32)], 0)


def _acc(ref, val):
    @pl.when(pl.program_id(0) == 0)
    def _():
        ref[...] = jnp.zeros_like(ref)
    ref[...] += val


def _full(shape):
    nd = len(shape)
    return pl.BlockSpec(shape, lambda g: (0,) * nd)


def _rows(shape):
    nd = len(shape)
    return pl.BlockSpec(shape, lambda g: (g,) + (0,) * (nd - 1))


def _k1(feat2, w_m1, ch, grid):
    def body(f_ref, w_ref, y_ref, st_ref):
        y = _mm(f_ref[...], w_ref[...])
        y_ref[...] = y
        _acc(st_ref, _stat2(y))

    r = feat2.shape[0]
    return pl.pallas_call(
        body,
        grid=(grid,),
        in_specs=[_rows((ch, 8)), _full((8, 8))],
        out_specs=[_rows((ch, 8)), _full((8, 8))],
        out_shape=[jax.ShapeDtypeStruct((r, 8), jnp.float32),
                   jax.ShapeDtypeStruct((8, 8), jnp.float32)],
    )(feat2, w_m1)


def _k2(y_m1, xyzp, bn1, ch, grid):
    def body(y_ref, x_ref, b_ref, t_ref):
        f_pc = _bn_apply(y_ref[...], b_ref[...])
        t_ref[...] = jnp.concatenate(
            [x_ref[...][:, 0:3], f_pc,
             jnp.zeros((y_ref.shape[0], 5), jnp.float32)], axis=1)

    r = y_m1.shape[0]
    return pl.pallas_call(
        body,
        grid=(grid,),
        in_specs=[_rows((ch, 8)), _rows((ch, 4)), _full((8, 8))],
        out_specs=_rows((ch, 16)),
        out_shape=jax.ShapeDtypeStruct((r, 16), jnp.float32),
    )(y_m1, xyzp, bn1)


def _k3(g1, table1, w_lc1, ch, grid, kk):
    def body(g_ref, t_ref, w_ref, y_ref, gd_ref, fd_ref, ms_ref, st_ref):
        g = g_ref[...]
        t = t_ref[...]
        c_xyz = t[:, 0:3]
        f_pc = t[:, 3:11]
        n_xyz = g[:, :, 0:3]
        fn1 = g[:, :, 3:11]
        rel = c_xyz[:, None, :] - n_xyz
        rel_a = jnp.arctan2(rel[:, :, 1], rel[:, :, 0])
        rel_xyd = jnp.sqrt(rel[:, :, 0] ** 2 + rel[:, :, 1] ** 2)
        rel_b = jnp.arctan2(rel[:, :, 2], rel_xyd)
        rel_dis = jnp.sqrt(jnp.sum(rel * rel, axis=2))
        gd_ref[...] = jnp.exp(-rel_dis)
        lv = jnp.max(rel_dis, axis=1, keepdims=True) ** 3
        nm = jnp.mean(n_xyz, axis=1)
        dv = c_xyz - nm
        dir_a = jnp.arctan2(dv[:, 1], dv[:, 0])[:, None]
        dir_b = jnp.arctan2(dv[:, 2],
                            jnp.sqrt(dv[:, 0] ** 2 + dv[:, 1] ** 2))[:, None]
        feats = [rel_a - dir_a, rel_b - dir_b, rel_dis,
                 jnp.broadcast_to(c_xyz[:, None, 0], rel_dis.shape),
                 jnp.broadcast_to(c_xyz[:, None, 1], rel_dis.shape),
                 jnp.broadcast_to(c_xyz[:, None, 2], rel_dis.shape),
                 n_xyz[:, :, 0], n_xyz[:, :, 1], n_xyz[:, :, 2]]
        w = w_ref[...]
        y = jnp.zeros(rel_dis.shape + (8,), jnp.float32)
        for j in range(9):
            y = y + feats[j][:, :, None] * w[:, j][None, None, :]
        y_ref[...] = y
        _acc(st_ref, _stat2(y))
        d = jnp.abs(f_pc[:, None, :] - fn1)
        fd_ref[...] = jnp.exp(-jnp.mean(d, axis=2))
        gv = jnp.sqrt(jnp.sum(c_xyz * c_xyz, axis=1, keepdims=True)) ** 3
        ratio = lv / gv
        ms_ref[...] = jnp.concatenate(
            [ratio, jnp.zeros((ratio.shape[0], 7), jnp.float32)], axis=1)

    r = table1.shape[0]
    return pl.pallas_call(
        body,
        grid=(grid,),
        in_specs=[_rows((ch, kk, 16)), _rows((ch, 16)), _full((8, 9))],
        out_specs=[_rows((ch, kk, 8)), _rows((ch, kk)), _rows((ch, kk)),
                   _rows((ch, 8)), _full((8, 8))],
        out_shape=[jax.ShapeDtypeStruct((r, kk, 8), jnp.float32),
                   jax.ShapeDtypeStruct((r, kk), jnp.float32),
                   jax.ShapeDtypeStruct((r, kk), jnp.float32),
                   jax.ShapeDtypeStruct((r, 8), jnp.float32),
                   jax.ShapeDtypeStruct((8, 8), jnp.float32)],
    )(g1, table1, w_lc1)


def _att_pool(fn, lr, gd, fd, w_fc):
    att = (gd[:, :, None] * w_fc[:, 0][None, None, :]
           + (fd * LAMDA)[:, :, None] * w_fc[:, 1][None, None, :]
           + _mm3(fn, w_fc[:, 2:10]) + _mm3(lr, w_fc[:, 10:18]))
    att = att - jnp.max(att, axis=1, keepdims=True)
    e = jnp.exp(att)
    att = e / jnp.sum(e, axis=1, keepdims=True)
    fs = jnp.concatenate([fn, lr], axis=2)
    return jnp.sum(fs * att, axis=1)


def _k4(y_lc1, g1, gd, fd, bn_lc1, w_fc, w_mlp, w_lc2, ch, grid, kk):
    def body(y_ref, g_ref, gd_ref, fd_ref, b_ref, wf_ref, wm_ref, wl_ref,
             yp_ref, yl2_ref, stp_ref, stl_ref):
        lr1 = _bn_apply(y_ref[...], b_ref[...])
        fn1 = g_ref[...][:, :, 3:11]
        pooled = _att_pool(fn1, lr1, gd_ref[...], fd_ref[...], wf_ref[...])
        y_p1 = _mm(pooled, wm_ref[...])
        yp_ref[...] = y_p1
        _acc(stp_ref, _stat2(y_p1))
        y_lc2 = _mm3(lr1, wl_ref[...])
        yl2_ref[...] = y_lc2
        _acc(stl_ref, _stat2(y_lc2))

    r = y_lc1.shape[0]
    return pl.pallas_call(
        body,
        grid=(grid,),
        in_specs=[_rows((ch, kk, 8)), _rows((ch, kk, 16)), _rows((ch, kk)),
                  _rows((ch, kk)), _full((8, 8)), _full((16, 18)),
                  _full((8, 16)), _full((8, 8))],
        out_specs=[_rows((ch, 8)), _rows((ch, kk, 8)),
                   _full((8, 8)), _full((8, 8))],
        out_shape=[jax.ShapeDtypeStruct((r, 8), jnp.float32),
                   jax.ShapeDtypeStruct((r, kk, 8), jnp.float32),
                   jax.ShapeDtypeStruct((8, 8), jnp.float32),
                   jax.ShapeDtypeStruct((8, 8), jnp.float32)],
    )(y_lc1, g1, gd, fd, bn_lc1, w_fc, w_mlp, w_lc2)


def _k5(y_p1, bn_p1, ch, grid):
    def body(y_ref, b_ref, t_ref):
        f = _bn_apply(y_ref[...], b_ref[...])
        t_ref[...] = jnp.concatenate(
            [f, jnp.zeros((f.shape[0], 8), jnp.float32)], axis=1)

    r = y_p1.shape[0]
    return pl.pallas_call(
        body,
        grid=(grid,),
        in_specs=[_rows((ch, 8)), _full((8, 8))],
        out_specs=_rows((ch, 16)),
        out_shape=jax.ShapeDtypeStruct((r, 16), jnp.float32),
    )(y_p1, bn_p1)


def _k6(g2, table2, y_lc2, bn_lc2, gd, w_fc, w_mlp, ch, grid, kk):
    def body(g_ref, t_ref, y_ref, b_ref, gd_ref, wf_ref, wm_ref,
             yp_ref, st_ref):
        lr2 = _bn_apply(y_ref[...], b_ref[...])
        fn2 = g_ref[...][:, :, 0:8]
        f_lc1 = t_ref[...][:, 0:8]
        d = jnp.abs(f_lc1[:, None, :] - fn2)
        fd2 = jnp.exp(-jnp.mean(d, axis=2))
        pooled = _att_pool(fn2, lr2, gd_ref[...], fd2, wf_ref[...])
        y_p2 = _mm(pooled, wm_ref[...])
        yp_ref[...] = y_p2
        _acc(st_ref, _stat2(y_p2))

    r = table2.shape[0]
    return pl.pallas_call(
        body,
        grid=(grid,),
        in_specs=[_rows((ch, kk, 16)), _rows((ch, 16)), _rows((ch, kk, 8)),
                  _full((8, 8)), _rows((ch, kk)), _full((16, 18)),
                  _full((16, 16))],
        out_specs=[_rows((ch, 16)), _full((8, 16))],
        out_shape=[jax.ShapeDtypeStruct((r, 16), jnp.float32),
                   jax.ShapeDtypeStruct((8, 16), jnp.float32)],
    )(g2, table2, y_lc2, bn_lc2, gd, w_fc, w_mlp)


def _k7(y_p2, bn_p2, feat2, xyzp, misc, w_m2, w_sc, w_m3, ch, grid):
    def body(y_ref, b_ref, f_ref, x_ref, m_ref, w2_ref, ws_ref, w3_ref,
             ym2_ref, ysc_ref, ym3_ref, s2_ref, ss_ref, s3_ref):
        f_lc = _bn_apply(y_ref[...], b_ref[...])
        y_m2 = _mm(f_lc, w2_ref[...])
        ym2_ref[...] = y_m2
        _acc(s2_ref, _stat2(y_m2))
        y_sc = _mm(f_ref[...], ws_ref[...])
        ysc_ref[...] = y_sc
        _acc(ss_ref, _stat2(y_sc))
        gc_in = jnp.concatenate(
            [x_ref[...][:, 0:3], m_ref[...][:, 0:1]], axis=1)
        y_m3 = _mm(gc_in, w3_ref[...])
        ym3_ref[...] = y_m3
        _acc(s3_ref, _stat2(y_m3))

    r = y_p2.shape[0]
    return pl.pallas_call(
        body,
        grid=(grid,),
        in_specs=[_rows((ch, 16)), _full((8, 16)), _rows((ch, 8)),
                  _rows((ch, 4)), _rows((ch, 8)), _full((32, 16)),
                  _full((32, 8)), _full((32, 4))],
        out_specs=[_rows((ch, 32)), _rows((ch, 32)), _rows((ch, 32)),
                   _full((8, 32)), _full((8, 32)), _full((8, 32))],
        out_shape=[jax.ShapeDtypeStruct((r, 32), jnp.float32)] * 3
        + [jax.ShapeDtypeStruct((8, 32), jnp.float32)] * 3,
    )(y_p2, bn_p2, feat2, xyzp, misc, w_m2, w_sc, w_m3)


def _k8(y_m2, y_sc, y_m3, bn_m2, bn_sc, bn_m3, w_m4, ch, grid):
    def body(a_ref, b_ref, c_ref, ba_ref, bb_ref, bc_ref, w_ref,
             y_ref, st_ref):
        a = _bn_apply(a_ref[...], ba_ref[...], relu=False)
        b = _bn_apply(b_ref[...], bb_ref[...], relu=False)
        c = _bn_apply(c_ref[...], bc_ref[...], relu=False)
        h = jnp.concatenate([a + b, c], axis=1)
        y = _mm(h, w_ref[...])
        y_ref[...] = y
        _acc(st_ref, _stat2(y))

    r = y_m2.shape[0]
    return pl.pallas_call(
        body,
        grid=(grid,),
        in_specs=[_rows((ch, 32))] * 3 + [_full((8, 32))] * 3
        + [_full((32, 64))],
        out_specs=[_rows((ch, 32)), _full((8, 32))],
        out_shape=[jax.ShapeDtypeStruct((r, 32), jnp.float32),
                   jax.ShapeDtypeStruct((8, 32), jnp.float32)],
    )(y_m2, y_sc, y_m3, bn_m2, bn_sc, bn_m3, w_m4)


def _k9(y_m4, bn_m4, ch, grid):
    def body(y_ref, b_ref, o_ref):
        o_ref[...] = _bn_apply(y_ref[...], b_ref[...])

    r = y_m4.shape[0]
    return pl.pallas_call(
        body,
        grid=(grid,),
        in_specs=[_rows((ch, 32)), _full((8, 32))],
        out_specs=_rows((ch, 32)),
        out_shape=jax.ShapeDtypeStruct((r, 32), jnp.float32),
    )(y_m4, bn_m4)


def kernel(feature, xyz, neigh_idx, params):
    p = params
    b, c_in, n, _ = feature.shape
    kk = neigh_idx.shape[-1]
    r = b * n
    m = r * kk
    ch = 4096
    grid = r // ch
    ch_k = 256
    grid_k = r // ch_k

    feat2 = feature[..., 0].transpose(0, 2, 1).reshape(r, c_in)
    xyzp = jnp.concatenate(
        [xyz, jnp.zeros((b, n, 1), jnp.float32)], axis=-1).reshape(r, 4)
    idx_flat = (neigh_idx.astype(jnp.int32)
                + (jnp.arange(b, dtype=jnp.int32) * n)[:, None, None]
                ).reshape(m)

    y_m1, st1 = _k1(feat2, p['W_m1'], ch, grid)
    bn1 = _finalize(st1, p['g_m1'], p['b_m1'], r)
    table1 = _k2(y_m1, xyzp, bn1, ch, grid)

    g1 = _sc_gather(table1, idx_flat).reshape(r, kk, 16)

    y_lc1, gd, fd1, misc, st_lc1 = _k3(g1, table1, p['W_lc1'], ch_k,
                                       grid_k, kk)
    bn_lc1 = _finalize(st_lc1, p['g_lc1'], p['b_lc1'], r * kk)

    y_p1, y_lc2, st_p1, st_lc2 = _k4(
        y_lc1, g1, gd, fd1, bn_lc1, p['W_p1_fc'], p['W_p1_mlp'],
        p['W_lc2'], ch_k, grid_k, kk)
    bn_p1 = _finalize(st_p1, p['g_p1'], p['b_p1'], r)
    bn_lc2 = _finalize(st_lc2, p['g_lc2'], p['b_lc2'], r * kk)

    table2 = _k5(y_p1, bn_p1, ch, grid)
    g2 = _sc_gather(table2, idx_flat).reshape(r, kk, 16)

    y_p2, st_p2 = _k6(g2, table2, y_lc2, bn_lc2, gd, p['W_p2_fc'],
                      p['W_p2_mlp'], ch_k, grid_k, kk)
    bn_p2 = _finalize(st_p2, p['g_p2'], p['b_p2'], r)

    y_m2, y_sc, y_m3, st_m2, st_sc, st_m3 = _k7(
        y_p2, bn_p2, feat2, xyzp, misc, p['W_m2'], p['W_sc'], p['W_m3'],
        ch, grid)
    bn_m2 = _finalize(st_m2, p['g_m2'], p['b_m2'], r)
    bn_sc = _finalize(st_sc, p['g_sc'], p['b_sc'], r)
    bn_m3 = _finalize(st_m3, p['g_m3'], p['b_m3'], r)

    y_m4, st_m4 = _k8(y_m2, y_sc, y_m3, bn_m2, bn_sc, bn_m3, p['W_m4'],
                      ch, grid)
    bn_m4 = _finalize(st_m4, p['g_m4'], p['b_m4'], r)

    out = _k9(y_m4, bn_m4, ch, grid)
    return out.reshape(b, n, 32).transpose(0, 2, 1)[..., None]
